# baseline (speedup 1.0000x reference)
import functools

import jax
import jax.numpy as jnp
from jax.experimental import pallas as pl
from jax.experimental.pallas import tpu as pltpu

NUM_EXPERTS = 8
D_MODEL = 1024
D_FF = 2048
F_TILE = 1024
NF = D_FF // F_TILE
B = 256
NBLK = 15
T = 1024


def _route_kernel(x_ref, wr_ref, pos1_ref, pos2_ref, p1_ref, p2_ref,
                  be_ref, valid_ref):
    x = x_ref[...]
    logits = jnp.dot(x, wr_ref[...], preferred_element_type=jnp.float32)
    eidx = jax.lax.broadcasted_iota(jnp.int32, logits.shape, 1)
    m1 = jnp.max(logits, axis=1, keepdims=True)
    i1 = jnp.min(jnp.where(logits == m1, eidx, NUM_EXPERTS), axis=1, keepdims=True)
    neg = jnp.finfo(jnp.float32).min
    masked = jnp.where(eidx == i1, neg, logits)
    m2 = jnp.max(masked, axis=1, keepdims=True)
    i2 = jnp.min(jnp.where(masked == m2, eidx, NUM_EXPERTS), axis=1, keepdims=True)
    p2 = 1.0 / (1.0 + jnp.exp(m1 - m2))
    p1 = 1.0 - p2

    sel1 = (eidx == i1)
    sel2 = (eidx == i2)
    A = (sel1 | sel2).astype(jnp.float32)
    t0 = jax.lax.broadcasted_iota(jnp.int32, (T, T), 0)
    t1 = jax.lax.broadcasted_iota(jnp.int32, (T, T), 1)
    L = (t1 < t0).astype(jnp.float32)
    r = jax.lax.dot_general(L, A, (((1,), (0,)), ((), ())),
                            preferred_element_type=jnp.float32)
    counts = jnp.sum(A, axis=0, keepdims=True)
    pc = jnp.ceil(counts / B) * B
    e0 = jax.lax.broadcasted_iota(jnp.int32, (NUM_EXPERTS, NUM_EXPERTS), 0)
    e1 = jax.lax.broadcasted_iota(jnp.int32, (NUM_EXPERTS, NUM_EXPERTS), 1)
    Lx = (e0 < e1).astype(jnp.float32)
    base = jnp.dot(pc, Lx, preferred_element_type=jnp.float32)

    r1 = jnp.sum(jnp.where(sel1, r, 0.0), axis=1, keepdims=True)
    b1 = jnp.sum(jnp.where(sel1, jnp.broadcast_to(base, r.shape), 0.0),
                 axis=1, keepdims=True)
    r2 = jnp.sum(jnp.where(sel2, r, 0.0), axis=1, keepdims=True)
    b2 = jnp.sum(jnp.where(sel2, jnp.broadcast_to(base, r.shape), 0.0),
                 axis=1, keepdims=True)
    pos1_ref[...] = (b1 + r1).astype(jnp.int32)
    pos2_ref[...] = (b2 + r2).astype(jnp.int32)
    p1_ref[...] = p1
    p2_ref[...] = p2

    seg_end = base + pc
    s0 = (jax.lax.broadcasted_iota(jnp.int32, (NBLK, 1), 0) * B
          ).astype(jnp.float32)
    be = jnp.sum((jnp.broadcast_to(seg_end, (NBLK, NUM_EXPERTS))
                  <= jnp.broadcast_to(s0, (NBLK, NUM_EXPERTS))).astype(jnp.float32),
                 axis=1, keepdims=True)
    erow = jax.lax.broadcasted_iota(
        jnp.int32, (1, NUM_EXPERTS), 1).astype(jnp.float32)
    emax = jnp.max(jnp.where(counts > 0, erow, 0.0))
    total = jnp.sum(pc)
    be_ref[...] = jnp.minimum(be, emax).astype(jnp.int32)
    valid_ref[...] = (s0 < total).astype(jnp.int32)


def _ffn_kernel(be_sref, val_sref, x_ref, wg_ref, wu_ref, wd_ref,
                pos1_ref, pos2_ref, p1_ref, p2_ref, out_ref, x_scr, y_scr):
    b = pl.program_id(0)
    f = pl.program_id(1)

    @pl.when((b == 0) & (f == 0))
    def _init():
        out_ref[...] = jnp.zeros_like(out_ref)

    @pl.when(val_sref[b] == 1)
    def _work():
        slot = (jax.lax.broadcasted_iota(jnp.int32, (T, B), 1) + b * B)
        eq1 = (jnp.broadcast_to(pos1_ref[...], (T, B)) == slot)
        eq2 = (jnp.broadcast_to(pos2_ref[...], (T, B)) == slot)

        @pl.when(f == 0)
        def _gather():
            PT = eq1.astype(jnp.float32) + eq2.astype(jnp.float32)
            x_scr[...] = jax.lax.dot_general(
                PT, x_ref[...], (((0,), (0,)), ((), ())),
                preferred_element_type=jnp.float32)

        x = x_scr[...]
        g = jnp.dot(x, wg_ref[0], preferred_element_type=jnp.float32)
        u = jnp.dot(x, wu_ref[0], preferred_element_type=jnp.float32)
        h = (g * jax.lax.logistic(g)) * u
        contrib = jnp.dot(h, wd_ref[0], preferred_element_type=jnp.float32)

        @pl.when(f == 0)
        def _y0():
            y_scr[...] = contrib

        @pl.when(f > 0)
        def _yacc():
            y_scr[...] += contrib

        @pl.when(f == NF - 1)
        def _combine():
            M = (p1_ref[...] * eq1.astype(jnp.float32)
                 + p2_ref[...] * eq2.astype(jnp.float32))
            out_ref[...] += jnp.dot(M, y_scr[...],
                                    preferred_element_type=jnp.float32)


def kernel(hidden_states, W_router, W_gate, W_up, W_down):
    batch, seq_len, hidden = hidden_states.shape
    flat = hidden_states.reshape(-1, hidden)

    pos1, pos2, p1, p2, be, valid = pl.pallas_call(
        _route_kernel,
        in_specs=[
            pl.BlockSpec((T, D_MODEL), lambda: (0, 0)),
            pl.BlockSpec((D_MODEL, NUM_EXPERTS), lambda: (0, 0)),
        ],
        out_specs=[
            pl.BlockSpec((T, 1), lambda: (0, 0)),
            pl.BlockSpec((T, 1), lambda: (0, 0)),
            pl.BlockSpec((T, 1), lambda: (0, 0)),
            pl.BlockSpec((T, 1), lambda: (0, 0)),
            pl.BlockSpec((NBLK, 1), lambda: (0, 0)),
            pl.BlockSpec((NBLK, 1), lambda: (0, 0)),
        ],
        out_shape=[
            jax.ShapeDtypeStruct((T, 1), jnp.int32),
            jax.ShapeDtypeStruct((T, 1), jnp.int32),
            jax.ShapeDtypeStruct((T, 1), jnp.float32),
            jax.ShapeDtypeStruct((T, 1), jnp.float32),
            jax.ShapeDtypeStruct((NBLK, 1), jnp.int32),
            jax.ShapeDtypeStruct((NBLK, 1), jnp.int32),
        ],
    )(flat, W_router)

    grid_spec = pltpu.PrefetchScalarGridSpec(
        num_scalar_prefetch=2,
        grid=(NBLK, NF),
        in_specs=[
            pl.BlockSpec((T, D_MODEL), lambda b, f, be_s, val_s: (0, 0)),
            pl.BlockSpec((1, D_MODEL, F_TILE),
                         lambda b, f, be_s, val_s: (be_s[b], 0, f)),
            pl.BlockSpec((1, D_MODEL, F_TILE),
                         lambda b, f, be_s, val_s: (be_s[b], 0, f)),
            pl.BlockSpec((1, F_TILE, D_MODEL),
                         lambda b, f, be_s, val_s: (be_s[b], f, 0)),
            pl.BlockSpec((T, 1), lambda b, f, be_s, val_s: (0, 0)),
            pl.BlockSpec((T, 1), lambda b, f, be_s, val_s: (0, 0)),
            pl.BlockSpec((T, 1), lambda b, f, be_s, val_s: (0, 0)),
            pl.BlockSpec((T, 1), lambda b, f, be_s, val_s: (0, 0)),
        ],
        out_specs=pl.BlockSpec((T, D_MODEL), lambda b, f, be_s, val_s: (0, 0)),
        scratch_shapes=[
            pltpu.VMEM((B, D_MODEL), jnp.float32),
            pltpu.VMEM((B, D_MODEL), jnp.float32),
        ],
    )

    out = pl.pallas_call(
        _ffn_kernel,
        grid_spec=grid_spec,
        out_shape=jax.ShapeDtypeStruct((T, D_MODEL), jnp.float32),
    )(be.reshape(NBLK), valid.reshape(NBLK),
      flat, W_gate, W_up, W_down, pos1, pos2, p1, p2)
    return out.reshape(batch, seq_len, hidden)


# speedup vs baseline: 1.1782x; 1.1782x over previous
import functools

import jax
import jax.numpy as jnp
from jax.experimental import pallas as pl
from jax.experimental.pallas import tpu as pltpu

NUM_EXPERTS = 8
D_MODEL = 1024
D_FF = 2048
F_TILE = 1024
NF = D_FF // F_TILE
B = 256
NBLK = 15
T = 1024


def _route_kernel(x_ref, wr_ref, pos1_ref, pos2_ref, p1_ref, p2_ref,
                  be_ref, valid_ref):
    x = x_ref[...]
    logits = jnp.dot(x, wr_ref[...], preferred_element_type=jnp.float32)
    eidx = jax.lax.broadcasted_iota(jnp.int32, logits.shape, 1)
    m1 = jnp.max(logits, axis=1, keepdims=True)
    i1 = jnp.min(jnp.where(logits == m1, eidx, NUM_EXPERTS), axis=1, keepdims=True)
    neg = jnp.finfo(jnp.float32).min
    masked = jnp.where(eidx == i1, neg, logits)
    m2 = jnp.max(masked, axis=1, keepdims=True)
    i2 = jnp.min(jnp.where(masked == m2, eidx, NUM_EXPERTS), axis=1, keepdims=True)
    p2 = 1.0 / (1.0 + jnp.exp(m1 - m2))
    p1 = 1.0 - p2

    sel1 = (eidx == i1)
    sel2 = (eidx == i2)
    A = (sel1 | sel2).astype(jnp.float32)
    t0 = jax.lax.broadcasted_iota(jnp.int32, (T, T), 0)
    t1 = jax.lax.broadcasted_iota(jnp.int32, (T, T), 1)
    L = (t1 < t0).astype(jnp.float32)
    r = jax.lax.dot_general(L, A, (((1,), (0,)), ((), ())),
                            preferred_element_type=jnp.float32)
    counts = jnp.sum(A, axis=0, keepdims=True)
    pc = jnp.ceil(counts / B) * B
    e0 = jax.lax.broadcasted_iota(jnp.int32, (NUM_EXPERTS, NUM_EXPERTS), 0)
    e1 = jax.lax.broadcasted_iota(jnp.int32, (NUM_EXPERTS, NUM_EXPERTS), 1)
    Lx = (e0 < e1).astype(jnp.float32)
    base = jnp.dot(pc, Lx, preferred_element_type=jnp.float32)

    r1 = jnp.sum(jnp.where(sel1, r, 0.0), axis=1, keepdims=True)
    b1 = jnp.sum(jnp.where(sel1, jnp.broadcast_to(base, r.shape), 0.0),
                 axis=1, keepdims=True)
    r2 = jnp.sum(jnp.where(sel2, r, 0.0), axis=1, keepdims=True)
    b2 = jnp.sum(jnp.where(sel2, jnp.broadcast_to(base, r.shape), 0.0),
                 axis=1, keepdims=True)
    pos1_ref[...] = (b1 + r1).astype(jnp.int32)
    pos2_ref[...] = (b2 + r2).astype(jnp.int32)
    p1_ref[...] = p1
    p2_ref[...] = p2

    seg_end = base + pc
    s0 = (jax.lax.broadcasted_iota(jnp.int32, (NBLK, 1), 0) * B
          ).astype(jnp.float32)
    be = jnp.sum((jnp.broadcast_to(seg_end, (NBLK, NUM_EXPERTS))
                  <= jnp.broadcast_to(s0, (NBLK, NUM_EXPERTS))).astype(jnp.float32),
                 axis=1, keepdims=True)
    erow = jax.lax.broadcasted_iota(
        jnp.int32, (1, NUM_EXPERTS), 1).astype(jnp.float32)
    emax = jnp.max(jnp.where(counts > 0, erow, 0.0))
    total = jnp.sum(pc)
    be_ref[...] = jnp.minimum(be, emax).astype(jnp.int32)
    valid_ref[...] = (s0 < total).astype(jnp.int32)


def _ffn_kernel(be_sref, val_sref, x_ref, wg_ref, wu_ref, wd_ref,
                pos1_ref, pos2_ref, p1_ref, p2_ref, out_ref):
    f = pl.program_id(0)
    b = pl.program_id(1)

    @pl.when((b == 0) & (f == 0))
    def _init():
        out_ref[...] = jnp.zeros_like(out_ref)

    @pl.when(val_sref[b] == 1)
    def _work():
        slot = (jax.lax.broadcasted_iota(jnp.int32, (T, B), 1) + b * B)
        eq1 = (jnp.broadcast_to(pos1_ref[...], (T, B)) == slot)
        eq2 = (jnp.broadcast_to(pos2_ref[...], (T, B)) == slot)

        PT = eq1.astype(jnp.float32) + eq2.astype(jnp.float32)
        x = jax.lax.dot_general(
            PT, x_ref[...], (((0,), (0,)), ((), ())),
            preferred_element_type=jnp.float32)
        g = jnp.dot(x, wg_ref[0], preferred_element_type=jnp.float32)
        u = jnp.dot(x, wu_ref[0], preferred_element_type=jnp.float32)
        h = (g * jax.lax.logistic(g)) * u
        y = jnp.dot(h, wd_ref[0], preferred_element_type=jnp.float32)
        M = (p1_ref[...] * eq1.astype(jnp.float32)
             + p2_ref[...] * eq2.astype(jnp.float32))
        out_ref[...] += jnp.dot(M, y, preferred_element_type=jnp.float32)


def kernel(hidden_states, W_router, W_gate, W_up, W_down):
    batch, seq_len, hidden = hidden_states.shape
    flat = hidden_states.reshape(-1, hidden)

    pos1, pos2, p1, p2, be, valid = pl.pallas_call(
        _route_kernel,
        in_specs=[
            pl.BlockSpec((T, D_MODEL), lambda: (0, 0)),
            pl.BlockSpec((D_MODEL, NUM_EXPERTS), lambda: (0, 0)),
        ],
        out_specs=[
            pl.BlockSpec((T, 1), lambda: (0, 0)),
            pl.BlockSpec((T, 1), lambda: (0, 0)),
            pl.BlockSpec((T, 1), lambda: (0, 0)),
            pl.BlockSpec((T, 1), lambda: (0, 0)),
            pl.BlockSpec((NBLK, 1), lambda: (0, 0)),
            pl.BlockSpec((NBLK, 1), lambda: (0, 0)),
        ],
        out_shape=[
            jax.ShapeDtypeStruct((T, 1), jnp.int32),
            jax.ShapeDtypeStruct((T, 1), jnp.int32),
            jax.ShapeDtypeStruct((T, 1), jnp.float32),
            jax.ShapeDtypeStruct((T, 1), jnp.float32),
            jax.ShapeDtypeStruct((NBLK, 1), jnp.int32),
            jax.ShapeDtypeStruct((NBLK, 1), jnp.int32),
        ],
    )(flat, W_router)

    grid_spec = pltpu.PrefetchScalarGridSpec(
        num_scalar_prefetch=2,
        grid=(NF, NBLK),
        in_specs=[
            pl.BlockSpec((T, D_MODEL), lambda f, b, be_s, val_s: (0, 0)),
            pl.BlockSpec((1, D_MODEL, F_TILE),
                         lambda f, b, be_s, val_s: (be_s[b], 0, f)),
            pl.BlockSpec((1, D_MODEL, F_TILE),
                         lambda f, b, be_s, val_s: (be_s[b], 0, f)),
            pl.BlockSpec((1, F_TILE, D_MODEL),
                         lambda f, b, be_s, val_s: (be_s[b], f, 0)),
            pl.BlockSpec((T, 1), lambda f, b, be_s, val_s: (0, 0)),
            pl.BlockSpec((T, 1), lambda f, b, be_s, val_s: (0, 0)),
            pl.BlockSpec((T, 1), lambda f, b, be_s, val_s: (0, 0)),
            pl.BlockSpec((T, 1), lambda f, b, be_s, val_s: (0, 0)),
        ],
        out_specs=pl.BlockSpec((T, D_MODEL), lambda f, b, be_s, val_s: (0, 0)),
    )

    out = pl.pallas_call(
        _ffn_kernel,
        grid_spec=grid_spec,
        out_shape=jax.ShapeDtypeStruct((T, D_MODEL), jnp.float32),
    )(be.reshape(NBLK), valid.reshape(NBLK),
      flat, W_gate, W_up, W_down, pos1, pos2, p1, p2)
    return out.reshape(batch, seq_len, hidden)
